# Initial kernel scaffold; baseline (speedup 1.0000x reference)
import jax
import jax.numpy as jnp
from jax.experimental import pallas as pl


def kernel(x, edge_attr, W1, b1, g1, bb1, W2, b2, eps, bn_g, bn_b, Wv1, bv1, gv1, bev1, Wv2, bv2, gv2, bev2, Wpred, bpred, edge_index, batch):
    raise NotImplementedError("write your pallas kernel here")


# speedup vs baseline: 1.0019x; 1.0019x over previous
import functools

import jax
import jax.numpy as jnp
from jax import lax
from jax.experimental import pallas as pl
from jax.experimental.pallas import tpu as pltpu
from jax.experimental.pallas import tpu_sc as plsc

_N = 10000
_E = 320000
_D = 128
_G = 512
_L = 5
_C = 10

_NW = 32
_CH = 32
_NCH = 316
_EPT = _CH * _NCH
_EPAD = _EPT * _NW
_NPAD = 10112
_RPT = _NPAD // 16

_NP2 = 10240
_RPW = _NP2 // 32
_GACC = 528
_GRT = 48

_BLK = 1000
_NB = _N // _BLK
_HALF = 5000


def _sc_edge_body(h_hbm, ea_hbm, src_hbm, dst_hbm, perm_hbm, out_hbm,
                  sidx, didx, eidx, hbuf, ebuf, zbuf, acc, sem):
    c = lax.axis_index("c")
    s = lax.axis_index("s")
    w = c * 16 + s

    zero16 = jnp.zeros((16,), jnp.float32)

    def zrow(r, carry):
        for j in range(8):
            zbuf[r, pl.ds(16 * j, 16)] = zero16
        return carry

    lax.fori_loop(0, _CH, zrow, 0)

    r0 = s * _RPT
    nz, rem = _RPT // _CH, _RPT % _CH
    for t in range(nz):
        pltpu.sync_copy(zbuf, acc.at[pl.ds(r0 + _CH * t, _CH)])
    if rem:
        pltpu.sync_copy(zbuf.at[pl.ds(0, rem)],
                        acc.at[pl.ds(r0 + _CH * nz, rem)])
    plsc.subcore_barrier()

    def chunk(k, carry):
        e0 = w * _EPT + k * _CH
        pltpu.sync_copy(src_hbm.at[pl.ds(e0, _CH)], sidx)
        pltpu.sync_copy(perm_hbm.at[pl.ds(e0, _CH)], eidx)
        pltpu.sync_copy(dst_hbm.at[pl.ds(e0, _CH)], didx.at[0])
        pltpu.async_copy(h_hbm.at[sidx], hbuf, sem).wait()
        pltpu.async_copy(ea_hbm.at[eidx], ebuf, sem).wait()

        def row(r, rc):
            for j in range(8):
                sl = pl.ds(16 * j, 16)
                hbuf[r, sl] = jnp.maximum(hbuf[r, sl] + ebuf[r, sl], 0.0)
            return rc

        lax.fori_loop(0, _CH, row, 0, unroll=2)
        pltpu.sync_copy(hbuf, acc.at[didx.at[0]], add=True)
        return carry

    lax.fori_loop(0, _NCH, chunk, 0)
    plsc.subcore_barrier()
    pltpu.sync_copy(acc.at[pl.ds(r0, _RPT)], out_hbm.at[c, pl.ds(r0, _RPT)])


@functools.lru_cache(maxsize=None)
def _sc_edge_kernel():
    return functools.partial(
        pl.kernel,
        out_type=jax.ShapeDtypeStruct((2, _NPAD, _D), jnp.float32),
        mesh=plsc.VectorSubcoreMesh(core_axis_name="c", subcore_axis_name="s"),
        scratch_types=[
            pltpu.VMEM((_CH,), jnp.int32),
            pltpu.VMEM((1, _CH), jnp.int32),
            pltpu.VMEM((_CH,), jnp.int32),
            pltpu.VMEM((_CH, _D), jnp.float32),
            pltpu.VMEM((_CH, _D), jnp.float32),
            pltpu.VMEM((_CH, _D), jnp.float32),
            pltpu.VMEM_SHARED((_NPAD, _D), jnp.float32),
            pltpu.SemaphoreType.DMA,
        ],
    )(_sc_edge_body)


def _sc_edge(h_in, ea_pad, src_s, dst_s, perm):
    return _sc_edge_kernel()(h_in, ea_pad, src_s, dst_s, perm)


def _sc_bsum_body(y_hbm, b_hbm, out_hbm, bidx, ybuf, zbuf, acc, sem):
    c = lax.axis_index("c")
    s = lax.axis_index("s")
    w = c * 16 + s

    zero16 = jnp.zeros((16,), jnp.float32)

    def zrow(r, carry):
        for j in range(8):
            zbuf[r, pl.ds(16 * j, 16)] = zero16
        return carry

    lax.fori_loop(0, _GRT, zrow, 0)

    @pl.when(s < 11)
    def _():
        r0 = s * _GRT
        pltpu.sync_copy(zbuf, acc.at[pl.ds(r0, _GRT)])

    plsc.subcore_barrier()

    for t in range(_RPW // _CH):
        n0 = w * _RPW + t * _CH
        pltpu.sync_copy(b_hbm.at[pl.ds(n0, _CH)], bidx.at[0])
        pltpu.async_copy(y_hbm.at[pl.ds(n0, _CH)], ybuf, sem).wait()
        pltpu.sync_copy(ybuf, acc.at[bidx.at[0]], add=True)

    plsc.subcore_barrier()

    @pl.when(s < 11)
    def _():
        g0 = s * 48
        pltpu.sync_copy(acc.at[pl.ds(g0, 48)], out_hbm.at[c, pl.ds(g0, 48)])


@functools.lru_cache(maxsize=None)
def _sc_bsum_kernel():
    return functools.partial(
        pl.kernel,
        out_type=jax.ShapeDtypeStruct((2, _GACC, _D), jnp.float32),
        mesh=plsc.VectorSubcoreMesh(core_axis_name="c", subcore_axis_name="s"),
        scratch_types=[
            pltpu.VMEM((1, _CH), jnp.int32),
            pltpu.VMEM((_CH, _D), jnp.float32),
            pltpu.VMEM((_GRT, _D), jnp.float32),
            pltpu.VMEM_SHARED((_GACC, _D), jnp.float32),
            pltpu.SemaphoreType.DMA,
        ],
    )(_sc_bsum_body)


def _sc_bsum(y_pad, batch_pad):
    return _sc_bsum_kernel()(y_pad, batch_pad)


def _onehot_t(batch_ref, blk):
    b = batch_ref[0]
    return (b == lax.broadcasted_iota(jnp.int32, (_G, blk), 0)).astype(
        jnp.float32)


def _p1_body(eps_ref, hin_ref, agg_ref, w1_ref, b1_ref, z1_ref):
    x0 = (1.0 + eps_ref[0, 0]) * hin_ref[...] + agg_ref[0] + agg_ref[1]
    z1_ref[...] = jnp.dot(x0, w1_ref[...]) + b1_ref[...]


def _sum2_body(z_ref, o_ref):
    o_ref[...] = jnp.sum(z_ref[...], axis=0, keepdims=True)


def _var2_body(s_ref, z_ref, o_ref):
    m = s_ref[...] * jnp.float32(1.0 / _N)
    c = z_ref[...] - m
    o_ref[0] = jnp.sum(c * c, axis=0, keepdims=True)


def _bn_from_stats(z, s_ref, q_ref, ga_ref, be_ref):
    inv = jnp.float32(1.0 / _N)
    m = s_ref[...] * inv
    v = (q_ref[0] + q_ref[1]) * inv
    return ga_ref[...] * (z - m) * lax.rsqrt(v + 1e-5) + be_ref[...]


def _p2_body(s1_ref, q1_ref, z1_ref, g1_ref, bb1_ref, w2_ref, b2_ref, z2_ref):
    t = jnp.maximum(
        _bn_from_stats(z1_ref[...], s1_ref, q1_ref, g1_ref, bb1_ref), 0.0)
    z2_ref[...] = jnp.dot(t, w2_ref[...]) + b2_ref[...]


def _p3_body(s2_ref, q2_ref, z2_ref, g_ref, b_ref, vn_ref, batch_ref, hn_ref):
    out = jnp.maximum(
        _bn_from_stats(z2_ref[...], s2_ref, q2_ref, g_ref, b_ref), 0.0)
    oht = _onehot_t(batch_ref, _BLK)
    vadd = lax.dot_general(oht, vn_ref[...], (((0,), (0,)), ((), ())),
                           precision=lax.Precision.HIGHEST)
    hn_ref[...] = out + vadd


def _p3f_body(s2_ref, q2_ref, z2_ref, g_ref, b_ref, batch_ref,
              h_ref, cnt_ref):
    i = pl.program_id(0)
    out = _bn_from_stats(z2_ref[...], s2_ref, q2_ref, g_ref, b_ref)
    h_ref[...] = out
    oht = _onehot_t(batch_ref, _BLK)
    ones = jnp.ones((_BLK, 1), jnp.float32)
    cnt = lax.dot_general(oht, ones, (((1,), (0,)), ((), ())),
                          precision=lax.Precision.HIGHEST)

    @pl.when(i == 0)
    def _():
        cnt_ref[...] = jnp.zeros_like(cnt_ref)

    cnt_ref[...] += cnt


def _vn_body(vp_ref, vn_ref, wv1_ref, bv1_ref, gv1_ref, bev1_ref,
             wv2_ref, bv2_ref, gv2_ref, bev2_ref, out_ref):
    def bn(h, ga, be):
        m = jnp.mean(h, axis=0, keepdims=True)
        c = h - m
        v = jnp.mean(c * c, axis=0, keepdims=True)
        return ga * (h - m) * lax.rsqrt(v + 1e-5) + be

    vt = (vp_ref[0, :_G, :] + vp_ref[1, :_G, :]) + vn_ref[...]
    z = jnp.dot(vt, wv1_ref[...]) + bv1_ref[...]
    t = jnp.maximum(bn(z, gv1_ref[...], bev1_ref[...]), 0.0)
    z2 = jnp.dot(t, wv2_ref[...]) + bv2_ref[...]
    out_ref[...] = jnp.maximum(bn(z2, gv2_ref[...], bev2_ref[...]), 0.0)


def _f_body(sp_ref, cnt_ref, wp_ref, bp_ref, out_ref):
    sums = sp_ref[0, :_G, :] + sp_ref[1, :_G, :]
    hg = sums / jnp.maximum(cnt_ref[...], 1.0)
    out_ref[...] = jnp.dot(hg, wp_ref[...]) + bp_ref[...]


def _full(shape):
    return pl.BlockSpec(shape, lambda i: tuple(0 for _ in shape))


def _seq(n):
    return pltpu.CompilerParams(dimension_semantics=("arbitrary",) * n)


def _p1_call(eps_l, h_in, agg, w1, b1):
    return pl.pallas_call(
        _p1_body,
        grid=(_NB,),
        in_specs=[
            pl.BlockSpec(memory_space=pltpu.SMEM),
            pl.BlockSpec((_BLK, _D), lambda i: (i, 0)),
            pl.BlockSpec((2, _BLK, _D), lambda i: (0, i, 0)),
            _full((_D, 2 * _D)),
            _full((1, 2 * _D)),
        ],
        out_specs=pl.BlockSpec((_BLK, 2 * _D), lambda i: (i, 0)),
        out_shape=jax.ShapeDtypeStruct((_N, 2 * _D), jnp.float32),
        compiler_params=_seq(1),
    )(eps_l, h_in, agg, w1, b1)


def _sum2_call(z):
    w = z.shape[1]
    return pl.pallas_call(
        _sum2_body,
        out_shape=jax.ShapeDtypeStruct((1, w), jnp.float32),
    )(z)


def _var2_call(z, s):
    w = z.shape[1]
    return pl.pallas_call(
        _var2_body,
        grid=(2,),
        in_specs=[
            _full((1, w)),
            pl.BlockSpec((_HALF, w), lambda i: (i, 0)),
        ],
        out_specs=pl.BlockSpec((1, 1, w), lambda i: (i, 0, 0)),
        out_shape=jax.ShapeDtypeStruct((2, 1, w), jnp.float32),
        compiler_params=_seq(1),
    )(s, z)


def _p2_call(s1, q1, z1, g1, bb1, w2, b2):
    return pl.pallas_call(
        _p2_body,
        grid=(_NB,),
        in_specs=[
            _full((1, 2 * _D)),
            _full((2, 1, 2 * _D)),
            pl.BlockSpec((_BLK, 2 * _D), lambda i: (i, 0)),
            _full((1, 2 * _D)),
            _full((1, 2 * _D)),
            _full((2 * _D, _D)),
            _full((1, _D)),
        ],
        out_specs=pl.BlockSpec((_BLK, _D), lambda i: (i, 0)),
        out_shape=jax.ShapeDtypeStruct((_N, _D), jnp.float32),
        compiler_params=_seq(1),
    )(s1, q1, z1, g1, bb1, w2, b2)


def _p3_call(s2, q2, z2, g, b, vn, batch3):
    return pl.pallas_call(
        _p3_body,
        grid=(_NB,),
        in_specs=[
            _full((1, _D)),
            _full((2, 1, _D)),
            pl.BlockSpec((_BLK, _D), lambda i: (i, 0)),
            _full((1, _D)),
            _full((1, _D)),
            _full((_G, _D)),
            pl.BlockSpec((1, 1, _BLK), lambda i: (i, 0, 0)),
        ],
        out_specs=pl.BlockSpec((_BLK, _D), lambda i: (i, 0)),
        out_shape=jax.ShapeDtypeStruct((_N, _D), jnp.float32),
        compiler_params=_seq(1),
    )(s2, q2, z2, g, b, vn, batch3)


def _p3f_call(s2, q2, z2, g, b, batch3):
    return pl.pallas_call(
        _p3f_body,
        grid=(_NB,),
        in_specs=[
            _full((1, _D)),
            _full((2, 1, _D)),
            pl.BlockSpec((_BLK, _D), lambda i: (i, 0)),
            _full((1, _D)),
            _full((1, _D)),
            pl.BlockSpec((1, 1, _BLK), lambda i: (i, 0, 0)),
        ],
        out_specs=[
            pl.BlockSpec((_BLK, _D), lambda i: (i, 0)),
            _full((_G, 1)),
        ],
        out_shape=[
            jax.ShapeDtypeStruct((_N, _D), jnp.float32),
            jax.ShapeDtypeStruct((_G, 1), jnp.float32),
        ],
        compiler_params=_seq(1),
    )(s2, q2, z2, g, b, batch3)


def _vn_call(vp, vn, wv1, bv1, gv1, bev1, wv2, bv2, gv2, bev2):
    return pl.pallas_call(
        _vn_body,
        out_shape=jax.ShapeDtypeStruct((_G, _D), jnp.float32),
    )(vp, vn, wv1, bv1, gv1, bev1, wv2, bv2, gv2, bev2)


def _f_call(sp, cnt, wp, bp):
    return pl.pallas_call(
        _f_body,
        out_shape=jax.ShapeDtypeStruct((_G, _C), jnp.float32),
    )(sp, cnt, wp, bp)


def kernel(x, edge_attr, W1, b1, g1, bb1, W2, b2, eps, bn_g, bn_b,
           Wv1, bv1, gv1, bev1, Wv2, bv2, gv2, bev2, Wpred, bpred,
           edge_index, batch):
    pad = _EPAD - _E
    src_pad = jnp.concatenate([edge_index[0], jnp.zeros((pad,), jnp.int32)])
    dst_pad = jnp.concatenate([edge_index[1], jnp.full((pad,), _N, jnp.int32)])
    ea_pad = jnp.concatenate(
        [edge_attr, jnp.zeros((pad, _D), jnp.float32)], axis=0)
    perm = jnp.argsort(dst_pad, stable=True).astype(jnp.int32)
    src_s = src_pad[perm]
    dst_s = dst_pad[perm]
    batch3 = batch.reshape(_NB, 1, _BLK)
    batch_pad = jnp.concatenate(
        [batch, jnp.full((_NP2 - _N,), _G, jnp.int32)])
    zrows = jnp.zeros((_NP2 - _N, _D), jnp.float32)

    h_in = x
    vn = jnp.zeros((_G, _D), jnp.float32)
    out = None
    for l in range(_L):
        agg = _sc_edge(h_in, ea_pad, src_s, dst_s, perm)
        z1 = _p1_call(eps[l].reshape(1, 1), h_in, agg,
                      W1[l], b1[l].reshape(1, -1))
        s1 = _sum2_call(z1)
        q1 = _var2_call(z1, s1)
        z2 = _p2_call(s1, q1, z1, g1[l].reshape(1, -1), bb1[l].reshape(1, -1),
                      W2[l], b2[l].reshape(1, -1))
        s2 = _sum2_call(z2)
        q2 = _var2_call(z2, s2)
        if l < _L - 1:
            vp = _sc_bsum(jnp.concatenate([h_in, zrows], axis=0), batch_pad)
            vn = _vn_call(
                vp, vn,
                Wv1[l], bv1[l].reshape(1, -1), gv1[l].reshape(1, -1),
                bev1[l].reshape(1, -1),
                Wv2[l], bv2[l].reshape(1, -1), gv2[l].reshape(1, -1),
                bev2[l].reshape(1, -1))
            h_in = _p3_call(s2, q2, z2, bn_g[l].reshape(1, -1),
                            bn_b[l].reshape(1, -1), vn, batch3)
        else:
            h_out, cnt = _p3f_call(s2, q2, z2, bn_g[l].reshape(1, -1),
                                   bn_b[l].reshape(1, -1), batch3)
            sp = _sc_bsum(jnp.concatenate([h_out, zrows], axis=0), batch_pad)
            out = _f_call(sp, cnt, Wpred, bpred.reshape(1, -1))
    return out
